# baseline (device time: 73965 ns/iter reference)
import jax
import jax.numpy as jnp
from jax import lax
from jax.experimental import pallas as pl
from jax.experimental.pallas import tpu as pltpu

N_DEV = 16
E_PER = 8
N_EXP = 128
N_TOK = 2048
H = 1024
HH = H // 2
CHUNK = N_TOK // N_DEV
N_STEP = N_DEV - 1

RING = (0, 4, 8, 12, 15, 11, 7, 3, 2, 6, 10, 14, 13, 9, 5, 1)


def kernel(x, router_W, route_idx, expert_W, shared_W):
    ring = jnp.asarray(RING, dtype=jnp.int32)
    my = lax.axis_index("i")
    r = jnp.argmax(ring == my).astype(jnp.int32)
    idx = jnp.arange(N_DEV, dtype=jnp.int32)
    right = ring[jnp.mod(r + 1, N_DEV)].reshape(1)
    left = ring[jnp.mod(r - 1, N_DEV)].reshape(1)
    fwd_sched = ring[jnp.mod(r - 1 - idx, N_DEV)]
    bwd_sched = ring[jnp.mod(r + 1 + idx, N_DEV)]

    def body(x_ref, router_ref, route_ref, w_ref, shared_ref,
             right_ref, left_ref, fsched_ref, bsched_ref, out_ref,
             psel_ref, commf_ref, commb_ref,
             sendf_sems, recvf_sems, sendb_sems, recvb_sems):
        me = lax.axis_index("i")
        rgt = right_ref[0]
        lft = left_ref[0]

        scores = jnp.dot(x_ref[:, :], router_ref[:, :],
                         preferred_element_type=jnp.float32)
        smax = jnp.max(scores, axis=1, keepdims=True)
        p = jnp.exp(scores - smax)
        denom = jnp.sum(p, axis=1, keepdims=True)
        cols = lax.broadcasted_iota(jnp.int32, (N_TOK, N_EXP), 1)
        psel_ref[:, :] = jnp.sum(jnp.where(cols == route_ref[:, :], p, 0.0),
                                 axis=1, keepdims=True) / denom

        def local_contrib(c, lo):
            rows = pl.ds(c * CHUNK, CHUNK)
            xb = x_ref[rows, :]
            rb = route_ref[rows, :]
            pb = psel_ref[rows, :]
            parts = []
            for ei in range(E_PER):
                ge = me * E_PER + ei
                coeff = jnp.where(rb == ge, pb, 0.0)
                parts.append(xb * coeff)
            xcat = jnp.concatenate(parts, axis=1)
            return jnp.dot(xcat, w_ref[:, lo:lo + HH],
                           preferred_element_type=jnp.float32)

        own_rows = pl.ds(me * CHUNK, CHUNK)
        xown = x_ref[own_rows, :]
        shared_out = jnp.dot(xown, shared_ref[:, :],
                             preferred_element_type=jnp.float32)

        commf_ref[0] = local_contrib(fsched_ref[0], 0).astype(jnp.bfloat16)
        commb_ref[0] = local_contrib(bsched_ref[0], HH).astype(jnp.bfloat16)

        barrier = pltpu.get_barrier_semaphore()
        for nbr in (lft, rgt):
            pl.semaphore_signal(barrier, inc=1, device_id=(nbr,),
                                device_id_type=pl.DeviceIdType.MESH)
        pl.semaphore_wait(barrier, 2)

        for s in range(N_STEP):
            rdma_f = pltpu.make_async_remote_copy(
                src_ref=commf_ref.at[s],
                dst_ref=commf_ref.at[s + 1],
                send_sem=sendf_sems.at[s],
                recv_sem=recvf_sems.at[s],
                device_id=(rgt,),
                device_id_type=pl.DeviceIdType.MESH,
            )
            rdma_b = pltpu.make_async_remote_copy(
                src_ref=commb_ref.at[s],
                dst_ref=commb_ref.at[s + 1],
                send_sem=sendb_sems.at[s],
                recv_sem=recvb_sems.at[s],
                device_id=(lft,),
                device_id_type=pl.DeviceIdType.MESH,
            )
            rdma_f.start()
            rdma_b.start()
            contrib_f = local_contrib(fsched_ref[s + 1], 0)
            contrib_b = local_contrib(bsched_ref[s + 1], HH)
            rdma_f.wait()
            rdma_b.wait()
            commf_ref[s + 1] = (
                commf_ref[s + 1].astype(jnp.float32) + contrib_f
            ).astype(jnp.bfloat16)
            commb_ref[s + 1] = (
                commb_ref[s + 1].astype(jnp.float32) + contrib_b
            ).astype(jnp.bfloat16)

        out_ref[:, 0:HH] = (
            commf_ref[N_STEP].astype(jnp.float32) + shared_out[:, 0:HH])
        out_ref[:, HH:H] = (
            commb_ref[N_STEP].astype(jnp.float32) + shared_out[:, HH:H])

    return pl.pallas_call(
        body,
        out_shape=jax.ShapeDtypeStruct((CHUNK, H), jnp.float32),
        in_specs=(
            [pl.BlockSpec(memory_space=pltpu.VMEM)] * 5
            + [pl.BlockSpec(memory_space=pltpu.SMEM)] * 4
        ),
        out_specs=pl.BlockSpec(memory_space=pltpu.VMEM),
        scratch_shapes=[
            pltpu.VMEM((N_TOK, 1), jnp.float32),
            pltpu.VMEM((N_DEV, CHUNK, HH), jnp.bfloat16),
            pltpu.VMEM((N_DEV, CHUNK, HH), jnp.bfloat16),
            pltpu.SemaphoreType.DMA((N_STEP,)),
            pltpu.SemaphoreType.DMA((N_STEP,)),
            pltpu.SemaphoreType.DMA((N_STEP,)),
            pltpu.SemaphoreType.DMA((N_STEP,)),
        ],
        compiler_params=pltpu.CompilerParams(collective_id=0),
    )(x, router_W, route_idx, expert_W.reshape(E_PER * 512, H), shared_W,
      right, left, fwd_sched, bwd_sched)


# device time: 62903 ns/iter; 1.1759x vs baseline; 1.1759x over previous
import jax
import jax.numpy as jnp
from jax import lax
from jax.experimental import pallas as pl
from jax.experimental.pallas import tpu as pltpu

N_DEV = 16
E_PER = 8
N_EXP = 128
N_TOK = 2048
H = 1024
HH = H // 2
CHUNK = N_TOK // N_DEV
N_STEP = N_DEV - 1
N_SUB = 4
RSUB = CHUNK // N_SUB

RING = (0, 4, 8, 12, 15, 11, 7, 3, 2, 6, 10, 14, 13, 9, 5, 1)


def kernel(x, router_W, route_idx, expert_W, shared_W):
    ring = jnp.asarray(RING, dtype=jnp.int32)
    my = lax.axis_index("i")
    r = jnp.argmax(ring == my).astype(jnp.int32)
    idx = jnp.arange(N_DEV, dtype=jnp.int32)
    right = ring[jnp.mod(r + 1, N_DEV)].reshape(1)
    left = ring[jnp.mod(r - 1, N_DEV)].reshape(1)
    fwd_sched = ring[jnp.mod(r - 1 - idx, N_DEV)]
    bwd_sched = ring[jnp.mod(r + 1 + idx, N_DEV)]

    def body(x_ref, router_ref, route_ref, w_ref, shared_ref,
             right_ref, left_ref, fsched_ref, bsched_ref, out_ref,
             psel_ref, w16_ref, commf_ref, commb_ref,
             sendf_sems, recvf_sems, sendb_sems, recvb_sems):
        me = lax.axis_index("i")
        rgt = right_ref[0]
        lft = left_ref[0]

        scores = jnp.dot(x_ref[:, :], router_ref[:, :],
                         preferred_element_type=jnp.float32)
        smax = jnp.max(scores, axis=1, keepdims=True)
        p = jnp.exp(scores - smax)
        denom = jnp.sum(p, axis=1, keepdims=True)
        cols = lax.broadcasted_iota(jnp.int32, (N_TOK, N_EXP), 1)
        psel_ref[:, :] = jnp.sum(jnp.where(cols == route_ref[:, :], p, 0.0),
                                 axis=1, keepdims=True) / denom

        w16_ref[:, :] = w_ref[:, :].astype(jnp.bfloat16)

        def local_contrib(c, lo):
            rows = pl.ds(c * CHUNK, CHUNK)
            xb = x_ref[rows, :]
            rb = route_ref[rows, :]
            pb = psel_ref[rows, :]
            parts = []
            for ei in range(E_PER):
                ge = me * E_PER + ei
                coeff = jnp.where(rb == ge, pb, 0.0)
                parts.append(xb * coeff)
            xcat = jnp.concatenate(parts, axis=1).astype(jnp.bfloat16)
            return jnp.dot(xcat, w16_ref[:, lo:lo + HH],
                           preferred_element_type=jnp.float32)

        own_rows = pl.ds(me * CHUNK, CHUNK)
        xown = x_ref[own_rows, :]
        shared_out = jnp.dot(xown, shared_ref[:, :],
                             preferred_element_type=jnp.float32)

        commf_ref[0] = local_contrib(fsched_ref[0], 0).astype(jnp.bfloat16)
        commb_ref[0] = local_contrib(bsched_ref[0], HH).astype(jnp.bfloat16)

        def make_pair(k, s):
            rsl = pl.ds(k * RSUB, RSUB)
            i = k * N_STEP + s
            rf = pltpu.make_async_remote_copy(
                src_ref=commf_ref.at[s, rsl],
                dst_ref=commf_ref.at[s + 1, rsl],
                send_sem=sendf_sems.at[i],
                recv_sem=recvf_sems.at[i],
                device_id=(rgt,),
                device_id_type=pl.DeviceIdType.MESH,
            )
            rb = pltpu.make_async_remote_copy(
                src_ref=commb_ref.at[s, rsl],
                dst_ref=commb_ref.at[s + 1, rsl],
                send_sem=sendb_sems.at[i],
                recv_sem=recvb_sems.at[i],
                device_id=(lft,),
                device_id_type=pl.DeviceIdType.MESH,
            )
            return rf, rb

        barrier = pltpu.get_barrier_semaphore()
        for nbr in (lft, rgt):
            pl.semaphore_signal(barrier, inc=1, device_id=(nbr,),
                                device_id_type=pl.DeviceIdType.MESH)
        pl.semaphore_wait(barrier, 2)

        inflight = [make_pair(k, 0) for k in range(N_SUB)]
        for rf, rb in inflight:
            rf.start()
            rb.start()

        for s in range(N_STEP):
            contrib_f = local_contrib(fsched_ref[s + 1], 0)
            contrib_b = local_contrib(bsched_ref[s + 1], HH)
            for k in range(N_SUB):
                rsl = pl.ds(k * RSUB, RSUB)
                csl = slice(k * RSUB, (k + 1) * RSUB)
                rf, rb = inflight[k]
                rf.wait()
                commf_ref[s + 1, rsl] = (
                    commf_ref[s + 1, rsl].astype(jnp.float32)
                    + contrib_f[csl, :]
                ).astype(jnp.bfloat16)
                rb.wait()
                commb_ref[s + 1, rsl] = (
                    commb_ref[s + 1, rsl].astype(jnp.float32)
                    + contrib_b[csl, :]
                ).astype(jnp.bfloat16)
                if s + 1 < N_STEP:
                    nf, nb = make_pair(k, s + 1)
                    nf.start()
                    nb.start()
                    inflight[k] = (nf, nb)

        out_ref[:, 0:HH] = (
            commf_ref[N_STEP].astype(jnp.float32) + shared_out[:, 0:HH])
        out_ref[:, HH:H] = (
            commb_ref[N_STEP].astype(jnp.float32) + shared_out[:, HH:H])

    return pl.pallas_call(
        body,
        out_shape=jax.ShapeDtypeStruct((CHUNK, H), jnp.float32),
        in_specs=(
            [pl.BlockSpec(memory_space=pltpu.VMEM)] * 5
            + [pl.BlockSpec(memory_space=pltpu.SMEM)] * 4
        ),
        out_specs=pl.BlockSpec(memory_space=pltpu.VMEM),
        scratch_shapes=[
            pltpu.VMEM((N_TOK, 1), jnp.float32),
            pltpu.VMEM((E_PER * 512, H), jnp.bfloat16),
            pltpu.VMEM((N_DEV, CHUNK, HH), jnp.bfloat16),
            pltpu.VMEM((N_DEV, CHUNK, HH), jnp.bfloat16),
            pltpu.SemaphoreType.DMA((N_SUB * N_STEP,)),
            pltpu.SemaphoreType.DMA((N_SUB * N_STEP,)),
            pltpu.SemaphoreType.DMA((N_SUB * N_STEP,)),
            pltpu.SemaphoreType.DMA((N_SUB * N_STEP,)),
        ],
        compiler_params=pltpu.CompilerParams(collective_id=0),
    )(x, router_W, route_idx, expert_W.reshape(E_PER * 512, H), shared_W,
      right, left, fwd_sched, bwd_sched)
